# baseline (speedup 1.0000x reference)
import functools

import jax
import jax.numpy as jnp
from jax import lax
from jax.experimental import pallas as pl
from jax.experimental.pallas import tpu as pltpu
from jax.experimental.pallas import tpu_sc as plsc

N_NODES = 10000
N_EDGES = 160000
D = 256
MAX_SEQ = 500

NS = 16
VR = 640
ROWS = NS * VR // 16
RPT = ROWS // NS
EPT = 10240
ESTRIDE = 9984
NG = 512
GPT = NG // NS
NB = NG + 16

_mesh = plsc.VectorSubcoreMesh(
    core_axis_name="c", subcore_axis_name="s", num_cores=1)


@functools.partial(
    pl.kernel,
    out_type=jax.ShapeDtypeStruct((NG, D), jnp.float32),
    mesh=_mesh,
    scratch_types=[
        pltpu.VMEM((2, EPT), jnp.int32),
        pltpu.VMEM((ROWS, 16), jnp.int32),
        pltpu.VMEM((RPT, 16), jnp.int32),
        pltpu.VMEM((16,), jnp.int32),
        pltpu.VMEM((16, 16), jnp.int32),
        pltpu.VMEM((128,), jnp.int32),
        pltpu.VMEM((128,), jnp.int32),
        pltpu.VMEM((128,), jnp.int32),
        pltpu.VMEM((NB,), jnp.int32),
        pltpu.VMEM((GPT,), jnp.int32),
        pltpu.VMEM((GPT, D), jnp.float32),
        pltpu.VMEM_SHARED((ROWS, 16), jnp.int32),
        pltpu.VMEM_SHARED((NB,), jnp.int32),
        pltpu.VMEM_SHARED((16, 16), jnp.int32),
        pltpu.SemaphoreType.DMA,
    ],
    compiler_params=pltpu.CompilerParams(needs_layout_passes=False),
)
def _sc_unique_gather(x_hbm, e_hbm, gx_hbm, edges_v, bitmap_v,
                      p_local, cnt_v, cnts_l, idx_chunk, val_chunk, riota_v,
                      zn_v, nidx_v, rows_v, pres_sh, nodes_sh, cnts_sh, sem):
    s = lax.axis_index("s")
    iota = lax.iota(jnp.int32, 16)
    zeros16 = jnp.zeros((16,), jnp.int32)
    ones16 = jnp.ones((16,), jnp.int32)

    with jax.named_scope("edges_dma"):
        pltpu.sync_copy(e_hbm.at[:, pl.ds(s * ESTRIDE, EPT)], edges_v)

    with jax.named_scope("zero"):
        def _zb(r, _):
            for u in range(8):
                bitmap_v[r * 8 + u] = zeros16
            return 0
        lax.fori_loop(0, ROWS // 8, _zb, 0)
        pltpu.sync_copy(bitmap_v.at[pl.ds(0, RPT)],
                        pres_sh.at[pl.ds(s * RPT, RPT)])

        @pl.when(s == 0)
        def _init_nodes():
            def _zn(r, _):
                zn_v[pl.ds(r * 16, 16)] = zeros16
                return 0
            lax.fori_loop(0, NB // 16, _zn, 0)
            pltpu.sync_copy(zn_v, nodes_sh)

    with jax.named_scope("scatter"):
        def _scat(i, _):
            for u in range(5):
                v = edges_v[0, pl.ds(i * 80 + u * 16, 16)]
                plsc.store_scatter(bitmap_v, [v >> 4, v & 15], ones16)
            return 0
        lax.fori_loop(0, EPT // 80, _scat, 0)

    plsc.subcore_barrier()

    with jax.named_scope("reduce"):
        for j in range(ROWS // 128):
            def _ri(t, _):
                riota_v[pl.ds(t * 16, 16)] = j * 128 + t * 16 + iota
                return 0
            lax.fori_loop(0, 8, _ri, 0)
            pltpu.sync_copy(bitmap_v.at[pl.ds(j * 128, 128)],
                            pres_sh.at[riota_v], add=True)

    plsc.subcore_barrier()

    with jax.named_scope("counts"):
        pltpu.sync_copy(pres_sh.at[pl.ds(s * RPT, RPT)], p_local)

        def _cnt(g, acc):
            return acc + plsc.all_reduce_population_count(p_local[g] > 0)
        cnt = lax.fori_loop(0, RPT, _cnt, zeros16)
        cnt_v[...] = cnt
        pltpu.sync_copy(cnt_v, cnts_sh.at[s])

    plsc.subcore_barrier()

    with jax.named_scope("compact"):
        pltpu.sync_copy(cnts_sh, cnts_l)
        counts_vec = plsc.load_gather(cnts_l, [iota, iota])
        base = jnp.sum(jnp.where(iota < s, counts_vec, 0))

        run = jnp.int32(0)
        for j in range(RPT // 8):
            def _grp(g2, run):
                g = j * 8 + g2
                p = p_local[g]
                m = p > 0
                mi = m.astype(jnp.int32)
                incl = plsc.cumsum(mi)
                ranks = base + run + incl - 1
                vals = s * VR + g * 16 + iota
                cond = m & (ranks < MAX_SEQ)
                idx_chunk[pl.ds(g2 * 16, 16)] = jnp.where(cond, ranks,
                                                          NG + iota)
                val_chunk[pl.ds(g2 * 16, 16)] = jnp.where(cond, vals, 0)
                return run + jnp.sum(mi)
            run = lax.fori_loop(0, 8, _grp, run)
            pltpu.sync_copy(val_chunk, nodes_sh.at[idx_chunk])

    plsc.subcore_barrier()

    with jax.named_scope("xgather"):
        pltpu.sync_copy(nodes_sh.at[pl.ds(s * GPT, GPT)], nidx_v)
        pltpu.async_copy(x_hbm.at[nidx_v], rows_v, sem).wait()
        pltpu.sync_copy(rows_v, gx_hbm.at[pl.ds(s * GPT, GPT)])


def _mm_body(gx_ref, w_ref, b_ref, o_ref):
    acc = lax.dot_general(gx_ref[...], w_ref[...], (((1,), (1,)), ((), ())),
                          preferred_element_type=jnp.float32)
    o_ref[...] = acc[:MAX_SEQ] + b_ref[...]


_mm = pl.pallas_call(
    _mm_body,
    out_shape=jax.ShapeDtypeStruct((MAX_SEQ, D), jnp.float32),
)


def kernel(x, edge_index, W, b):
    gx = _sc_unique_gather(x, edge_index)
    return _mm(gx, W, b.reshape(1, D))[None]


# speedup vs baseline: 1.0998x; 1.0998x over previous
import functools

import jax
import jax.numpy as jnp
from jax import lax
from jax.experimental import pallas as pl
from jax.experimental.pallas import tpu as pltpu
from jax.experimental.pallas import tpu_sc as plsc

N_NODES = 10000
N_EDGES = 160000
D = 256
MAX_SEQ = 500

NS = 16
VR = 640
ROWS = NS * VR // 16
RPT = ROWS // NS
EPT = 10240
ESTRIDE = 9984
NG = 512
GPT = NG // NS
NB = NG + 16

_mesh = plsc.VectorSubcoreMesh(
    core_axis_name="c", subcore_axis_name="s", num_cores=1)


@functools.partial(
    pl.kernel,
    out_type=jax.ShapeDtypeStruct((NG, D), jnp.float32),
    mesh=_mesh,
    scratch_types=[
        pltpu.VMEM((2, EPT), jnp.int32),
        pltpu.VMEM((ROWS, 16), jnp.int32),
        pltpu.VMEM((RPT, 16), jnp.int32),
        pltpu.VMEM((16,), jnp.int32),
        pltpu.VMEM((16, 16), jnp.int32),
        pltpu.VMEM((128,), jnp.int32),
        pltpu.VMEM((128,), jnp.int32),
        pltpu.VMEM((128,), jnp.int32),
        pltpu.VMEM((NB,), jnp.int32),
        pltpu.VMEM((GPT,), jnp.int32),
        pltpu.VMEM((GPT, D), jnp.float32),
        pltpu.VMEM_SHARED((ROWS, 16), jnp.int32),
        pltpu.VMEM_SHARED((NB,), jnp.int32),
        pltpu.VMEM_SHARED((16, 16), jnp.int32),
        pltpu.SemaphoreType.DMA,
    ],
    compiler_params=pltpu.CompilerParams(needs_layout_passes=False),
)
def _sc_unique_gather(x_hbm, e_hbm, gx_hbm, edges_v, bitmap_v,
                      p_local, cnt_v, cnts_l, idx_chunk, val_chunk, riota_v,
                      zn_v, nidx_v, rows_v, pres_sh, nodes_sh, cnts_sh, sem):
    s = lax.axis_index("s")
    iota = lax.iota(jnp.int32, 16)
    zeros16 = jnp.zeros((16,), jnp.int32)
    ones16 = jnp.ones((16,), jnp.int32)

    with jax.named_scope("edges_dma"):
        edma = pltpu.async_copy(e_hbm.at[:, pl.ds(s * ESTRIDE, EPT)],
                                edges_v, sem)

    with jax.named_scope("zero"):
        def _zb(r, _):
            for u in range(8):
                bitmap_v[r * 8 + u] = zeros16
            return 0
        lax.fori_loop(0, ROWS // 8, _zb, 0)
        pltpu.sync_copy(bitmap_v.at[pl.ds(0, RPT)],
                        pres_sh.at[pl.ds(s * RPT, RPT)])

        @pl.when(s == 0)
        def _init_nodes():
            def _zn(r, _):
                zn_v[pl.ds(r * 16, 16)] = zeros16
                return 0
            lax.fori_loop(0, NB // 16, _zn, 0)
            pltpu.sync_copy(zn_v, nodes_sh)

    with jax.named_scope("scatter"):
        edma.wait()

        def _scat(i, _):
            for u in range(8):
                v = edges_v[0, pl.ds(i * 128 + u * 16, 16)]
                plsc.store_scatter(bitmap_v, [v >> 4, v & 15], ones16)
            return 0
        lax.fori_loop(0, EPT // 128, _scat, 0)

    plsc.subcore_barrier()

    with jax.named_scope("reduce"):
        for j in range(ROWS // 128):
            def _ri(t, _):
                riota_v[pl.ds(t * 16, 16)] = j * 128 + t * 16 + iota
                return 0
            lax.fori_loop(0, 8, _ri, 0)
            pltpu.sync_copy(bitmap_v.at[pl.ds(j * 128, 128)],
                            pres_sh.at[riota_v], add=True)

    plsc.subcore_barrier()

    with jax.named_scope("counts"):
        pltpu.sync_copy(pres_sh.at[pl.ds(s * RPT, RPT)], p_local)

        def _cnt(g, acc):
            return acc + plsc.all_reduce_population_count(p_local[g] > 0)
        cnt = lax.fori_loop(0, RPT, _cnt, zeros16)
        cnt_v[...] = cnt
        pltpu.sync_copy(cnt_v, cnts_sh.at[s])

    plsc.subcore_barrier()

    with jax.named_scope("compact"):
        pltpu.sync_copy(cnts_sh, cnts_l)
        counts_vec = plsc.load_gather(cnts_l, [iota, iota])
        base = jnp.sum(jnp.where(iota < s, counts_vec, 0))

        run = jnp.int32(0)
        for j in range(RPT // 8):
            def _grp(g2, run):
                g = j * 8 + g2
                p = p_local[g]
                m = p > 0
                mi = m.astype(jnp.int32)
                incl = plsc.cumsum(mi)
                ranks = base + run + incl - 1
                vals = s * VR + g * 16 + iota
                cond = m & (ranks < MAX_SEQ)
                idx_chunk[pl.ds(g2 * 16, 16)] = jnp.where(cond, ranks,
                                                          NG + iota)
                val_chunk[pl.ds(g2 * 16, 16)] = jnp.where(cond, vals, 0)
                return run + jnp.sum(mi)
            run = lax.fori_loop(0, 8, _grp, run)
            pltpu.sync_copy(val_chunk, nodes_sh.at[idx_chunk])

    plsc.subcore_barrier()

    with jax.named_scope("xgather"):
        pltpu.sync_copy(nodes_sh.at[pl.ds(s * GPT, GPT)], nidx_v)
        pltpu.async_copy(x_hbm.at[nidx_v], rows_v, sem).wait()
        pltpu.sync_copy(rows_v, gx_hbm.at[pl.ds(s * GPT, GPT)])


def _mm_body(gx_ref, w_ref, b_ref, o_ref):
    acc = lax.dot_general(gx_ref[...], w_ref[...], (((1,), (1,)), ((), ())),
                          preferred_element_type=jnp.float32)
    h = acc[:MAX_SEQ] + b_ref[...]
    o_ref[...] = h.reshape(2 * MAX_SEQ, 128)


_mm = pl.pallas_call(
    _mm_body,
    out_shape=jax.ShapeDtypeStruct((2 * MAX_SEQ, 128), jnp.float32),
)


def kernel(x, edge_index, W, b):
    gx = _sc_unique_gather(x, edge_index)
    return _mm(gx, W, b.reshape(1, D)).reshape(1, MAX_SEQ, D)


# speedup vs baseline: 1.2725x; 1.1570x over previous
import functools

import jax
import jax.numpy as jnp
from jax import lax
from jax.experimental import pallas as pl
from jax.experimental.pallas import tpu as pltpu
from jax.experimental.pallas import tpu_sc as plsc

N_NODES = 10000
N_EDGES = 160000
D = 256
MAX_SEQ = 500

NS = 16
VR = 640
ROWS = NS * VR // 16
RPT = ROWS // NS
EPT = 10240
ESTRIDE = 9984
NG = 512
GPT = NG // NS
NB = NG + 16

_mesh = plsc.VectorSubcoreMesh(
    core_axis_name="c", subcore_axis_name="s", num_cores=1)


@functools.partial(
    pl.kernel,
    out_type=jax.ShapeDtypeStruct((NG, D), jnp.float32),
    mesh=_mesh,
    scratch_types=[
        pltpu.VMEM((2, EPT), jnp.int32),
        pltpu.VMEM((ROWS, 16), jnp.int32),
        pltpu.VMEM((RPT, 16), jnp.int32),
        pltpu.VMEM((16,), jnp.int32),
        pltpu.VMEM((16, 16), jnp.int32),
        pltpu.VMEM((128,), jnp.int32),
        pltpu.VMEM((128,), jnp.int32),
        pltpu.VMEM((128,), jnp.int32),
        pltpu.VMEM((NB,), jnp.int32),
        pltpu.VMEM((GPT,), jnp.int32),
        pltpu.VMEM((GPT, D), jnp.float32),
        pltpu.VMEM_SHARED((ROWS, 16), jnp.int32),
        pltpu.VMEM_SHARED((NB,), jnp.int32),
        pltpu.VMEM_SHARED((16, 16), jnp.int32),
        pltpu.SemaphoreType.DMA,
    ],
    compiler_params=pltpu.CompilerParams(needs_layout_passes=False),
)
def _sc_unique_gather(x_hbm, e_hbm, gx_hbm, edges_v, bitmap_v,
                      p_local, cnt_v, cnts_l, idx_chunk, val_chunk, riota_v,
                      zn_v, nidx_v, rows_v, pres_sh, nodes_sh, cnts_sh, sem):
    s = lax.axis_index("s")
    iota = lax.iota(jnp.int32, 16)
    zeros16 = jnp.zeros((16,), jnp.int32)
    ones16 = jnp.ones((16,), jnp.int32)

    with jax.named_scope("edges_dma"):
        edma = pltpu.async_copy(e_hbm.at[:, pl.ds(s * ESTRIDE, EPT)],
                                edges_v, sem)

    with jax.named_scope("zero"):
        def _zb(r, _):
            for u in range(8):
                bitmap_v[r * 8 + u] = zeros16
            return 0
        lax.fori_loop(0, ROWS // 8, _zb, 0)
        pltpu.sync_copy(bitmap_v.at[pl.ds(0, RPT)],
                        pres_sh.at[pl.ds(s * RPT, RPT)])

        @pl.when(s == 0)
        def _init_nodes():
            def _zn(r, _):
                zn_v[pl.ds(r * 16, 16)] = zeros16
                return 0
            lax.fori_loop(0, NB // 16, _zn, 0)
            pltpu.sync_copy(zn_v, nodes_sh)

    with jax.named_scope("scatter"):
        edma.wait()

        @plsc.parallel_loop(0, EPT // 16, unroll=8)
        def _scat(i):
            v = edges_v[0, pl.ds(i * 16, 16)]
            plsc.store_scatter(bitmap_v, [v >> 4, v & 15], ones16)

    plsc.subcore_barrier()

    with jax.named_scope("reduce"):
        for j in range(ROWS // 128):
            def _ri(t, _):
                riota_v[pl.ds(t * 16, 16)] = j * 128 + t * 16 + iota
                return 0
            lax.fori_loop(0, 8, _ri, 0)
            pltpu.sync_copy(bitmap_v.at[pl.ds(j * 128, 128)],
                            pres_sh.at[riota_v], add=True)

    plsc.subcore_barrier()

    with jax.named_scope("counts"):
        pltpu.sync_copy(pres_sh.at[pl.ds(s * RPT, RPT)], p_local)

        def _cnt(g, acc):
            return acc + jnp.minimum(p_local[g], 1)
        acc = lax.fori_loop(0, RPT, _cnt, zeros16)
        cnt_v[...] = jnp.broadcast_to(jnp.sum(acc), (16,))
        pltpu.sync_copy(cnt_v, cnts_sh.at[s])

    plsc.subcore_barrier()

    with jax.named_scope("compact"):
        pltpu.sync_copy(cnts_sh, cnts_l)
        counts_vec = plsc.load_gather(cnts_l, [iota, iota])
        base = jnp.sum(jnp.where(iota < s, counts_vec, 0))

        run = jnp.int32(0)
        for j in range(RPT // 8):
            def _grp(g2, run):
                g = j * 8 + g2
                p = p_local[g]
                m = p > 0
                mi = m.astype(jnp.int32)
                incl = plsc.cumsum(mi)
                ranks = base + run + incl - 1
                vals = s * VR + g * 16 + iota
                cond = m & (ranks < MAX_SEQ)
                idx_chunk[pl.ds(g2 * 16, 16)] = jnp.where(cond, ranks,
                                                          NG + iota)
                val_chunk[pl.ds(g2 * 16, 16)] = jnp.where(cond, vals, 0)
                return run + incl[15]
            run = lax.fori_loop(0, 8, _grp, run)
            pltpu.sync_copy(val_chunk, nodes_sh.at[idx_chunk])

    plsc.subcore_barrier()

    with jax.named_scope("xgather"):
        pltpu.sync_copy(nodes_sh.at[pl.ds(s * GPT, GPT)], nidx_v)
        pltpu.async_copy(x_hbm.at[nidx_v], rows_v, sem).wait()
        pltpu.sync_copy(rows_v, gx_hbm.at[pl.ds(s * GPT, GPT)])


def _mm_body(gx_ref, w_ref, b_ref, o_ref):
    acc = lax.dot_general(gx_ref[...], w_ref[...], (((1,), (1,)), ((), ())),
                          preferred_element_type=jnp.float32)
    h = acc[:MAX_SEQ] + b_ref[...]
    o_ref[...] = h.reshape(2 * MAX_SEQ, 128)


_mm = pl.pallas_call(
    _mm_body,
    out_shape=jax.ShapeDtypeStruct((2 * MAX_SEQ, 128), jnp.float32),
)


def kernel(x, edge_index, W, b):
    gx = _sc_unique_gather(x, edge_index)
    return _mm(gx, W, b.reshape(1, D)).reshape(1, MAX_SEQ, D)


# speedup vs baseline: 1.2871x; 1.0115x over previous
import functools

import jax
import jax.numpy as jnp
from jax import lax
from jax.experimental import pallas as pl
from jax.experimental.pallas import tpu as pltpu
from jax.experimental.pallas import tpu_sc as plsc

N_NODES = 10000
N_EDGES = 160000
D = 256
MAX_SEQ = 500

NS = 16
VR = 640
ROWS = NS * VR // 16
RPT = ROWS // NS
EPT = 10240
ESTRIDE = 9984
NG = 512
GPT = NG // NS
NB = NG + 16

_mesh = plsc.VectorSubcoreMesh(
    core_axis_name="c", subcore_axis_name="s", num_cores=1)


@functools.partial(
    pl.kernel,
    out_type=jax.ShapeDtypeStruct((NG, D), jnp.float32),
    mesh=_mesh,
    scratch_types=[
        pltpu.VMEM((2, EPT), jnp.int32),
        pltpu.VMEM((ROWS, 16), jnp.int32),
        pltpu.VMEM((RPT, 16), jnp.int32),
        pltpu.VMEM((16,), jnp.int32),
        pltpu.VMEM((16, 16), jnp.int32),
        pltpu.VMEM((RPT // 8, 128), jnp.int32),
        pltpu.VMEM((RPT // 8, 128), jnp.int32),
        pltpu.VMEM((ROWS // 128, 128), jnp.int32),
        pltpu.VMEM((NB,), jnp.int32),
        pltpu.VMEM((GPT,), jnp.int32),
        pltpu.VMEM((GPT, D), jnp.float32),
        pltpu.VMEM_SHARED((ROWS, 16), jnp.int32),
        pltpu.VMEM_SHARED((NB,), jnp.int32),
        pltpu.VMEM_SHARED((16, 16), jnp.int32),
        pltpu.SemaphoreType.DMA,
    ],
    compiler_params=pltpu.CompilerParams(needs_layout_passes=False),
)
def _sc_unique_gather(x_hbm, e_hbm, gx_hbm, edges_v, bitmap_v,
                      p_local, cnt_v, cnts_l, idx_chunk, val_chunk, riota_v,
                      zn_v, nidx_v, rows_v, pres_sh, nodes_sh, cnts_sh, sem):
    s = lax.axis_index("s")
    iota = lax.iota(jnp.int32, 16)
    zeros16 = jnp.zeros((16,), jnp.int32)
    ones16 = jnp.ones((16,), jnp.int32)

    with jax.named_scope("edges_dma"):
        edma = pltpu.async_copy(e_hbm.at[:, pl.ds(s * ESTRIDE, EPT)],
                                edges_v, sem)

    with jax.named_scope("zero"):
        @plsc.parallel_loop(0, ROWS, unroll=8)
        def _zb(r):
            bitmap_v[r] = zeros16

        @plsc.parallel_loop(0, ROWS // 16, unroll=4)
        def _ri(g):
            riota_v[g >> 3, pl.ds((g & 7) * 16, 16)] = g * 16 + iota

        pltpu.sync_copy(bitmap_v.at[pl.ds(0, RPT)],
                        pres_sh.at[pl.ds(s * RPT, RPT)])

        @pl.when(s == 0)
        def _init_nodes():
            def _zn(r, _):
                zn_v[pl.ds(r * 16, 16)] = zeros16
                return 0
            lax.fori_loop(0, NB // 16, _zn, 0)
            pltpu.sync_copy(zn_v, nodes_sh)

    with jax.named_scope("scatter"):
        edma.wait()

        @plsc.parallel_loop(0, EPT // 16, unroll=8)
        def _scat(i):
            v = edges_v[0, pl.ds(i * 16, 16)]
            plsc.store_scatter(bitmap_v, [v >> 4, v & 15], ones16)

    plsc.subcore_barrier()

    with jax.named_scope("reduce"):
        copies = [
            pltpu.async_copy(bitmap_v.at[pl.ds(j * 128, 128)],
                             pres_sh.at[riota_v.at[j]], sem, add=True)
            for j in range(ROWS // 128)
        ]
        for c in copies:
            c.wait()

    plsc.subcore_barrier()

    with jax.named_scope("counts"):
        pltpu.sync_copy(pres_sh.at[pl.ds(s * RPT, RPT)], p_local)

        def _cnt(g, acc):
            return acc + jnp.minimum(p_local[g], 1)
        acc = lax.fori_loop(0, RPT, _cnt, zeros16)
        cnt_v[...] = jnp.broadcast_to(jnp.sum(acc), (16,))
        pltpu.sync_copy(cnt_v, cnts_sh.at[s])

    plsc.subcore_barrier()

    with jax.named_scope("compact"):
        pltpu.sync_copy(cnts_sh, cnts_l)
        counts_vec = plsc.load_gather(cnts_l, [iota, iota])
        base = jnp.sum(jnp.where(iota < s, counts_vec, 0))

        run = jnp.int32(0)
        copies = []
        for j in range(RPT // 8):
            def _grp(g2, run):
                g = j * 8 + g2
                p = p_local[g]
                m = p > 0
                mi = m.astype(jnp.int32)
                incl = plsc.cumsum(mi)
                ranks = base + run + incl - 1
                vals = s * VR + g * 16 + iota
                cond = m & (ranks < MAX_SEQ)
                idx_chunk[j, pl.ds(g2 * 16, 16)] = jnp.where(cond, ranks,
                                                             NG + iota)
                val_chunk[j, pl.ds(g2 * 16, 16)] = jnp.where(cond, vals, 0)
                return run + incl[15]
            run = lax.fori_loop(0, 8, _grp, run)
            copies.append(pltpu.async_copy(val_chunk.at[j],
                                           nodes_sh.at[idx_chunk.at[j]], sem))
        for c in copies:
            c.wait()

    plsc.subcore_barrier()

    with jax.named_scope("xgather"):
        pltpu.sync_copy(nodes_sh.at[pl.ds(s * GPT, GPT)], nidx_v)
        pltpu.async_copy(x_hbm.at[nidx_v], rows_v, sem).wait()
        pltpu.sync_copy(rows_v, gx_hbm.at[pl.ds(s * GPT, GPT)])


def _mm_body(gx_ref, w_ref, b_ref, o_ref):
    acc = lax.dot_general(gx_ref[...], w_ref[...], (((1,), (1,)), ((), ())),
                          preferred_element_type=jnp.float32)
    h = acc[:MAX_SEQ] + b_ref[...]
    o_ref[...] = h.reshape(2 * MAX_SEQ, 128)


_mm = pl.pallas_call(
    _mm_body,
    out_shape=jax.ShapeDtypeStruct((2 * MAX_SEQ, 128), jnp.float32),
)


def kernel(x, edge_index, W, b):
    gx = _sc_unique_gather(x, edge_index)
    return _mm(gx, W, b.reshape(1, D)).reshape(1, MAX_SEQ, D)
